# baseline (speedup 1.0000x reference)
import functools
import math

import jax
import jax.numpy as jnp
from jax import lax
from jax.experimental import pallas as pl
from jax.experimental.pallas import tpu as pltpu
from jax.experimental.pallas import tpu_sc as plsc

_N = 1000
_NPAD = 1024
_C = 256
_CP = _C // 2
_H = 64
_W = 64
_P = 7
_NB = _P * _P
_NCLS = 91
_CPAD = 128
_FC = 1024
_SCALE = 0.0625
_PS = 14
_NR = 200
_G1 = 96
_G2 = 104
_NW = 32
_RPW = _NPAD // _NW
_MAGIC = 8388608.0
_LOGK = math.log(1000.0 / 16)


def _rint(t):
    return (t + _MAGIC) - _MAGIC


def _table_body(feat_ref, out_ref):
    xt = jnp.transpose(feat_ref[...])
    out_ref[...] = xt.astype(jnp.bfloat16)


def _prep_table(feat):
    return pl.pallas_call(
        _table_body,
        out_shape=jax.ShapeDtypeStruct((_H * _W, _C), jnp.bfloat16),
    )(feat)


def _w6_body(w_ref, out_ref):
    out_ref[...] = w_ref[...].reshape(1, _C, 8, 128).astype(jnp.bfloat16)


def _prep_w6(w6v):
    return pl.pallas_call(
        _w6_body,
        grid=(_NB,),
        in_specs=[pl.BlockSpec((_C, 1, 8, 128), lambda p: (0, p, 0, 0))],
        out_specs=pl.BlockSpec((1, _C, 8, 128), lambda p: (p, 0, 0, 0)),
        out_shape=jax.ShapeDtypeStruct((_NB, _C, 8, 128), jnp.bfloat16),
        compiler_params=pltpu.CompilerParams(
            dimension_semantics=("arbitrary",)),
    )(w6v)


def _sc_pool_body(table_hbm, prop_hbm, out_hbm, prop_v, idx_v,
                  rows0, rows1, pooled0, pooled1, gs0, gs1, ws0, ws1):
    cid = lax.axis_index("c")
    sid = lax.axis_index("s")
    wid = sid * 2 + cid
    base = wid * _RPW

    pltpu.sync_copy(prop_hbm.at[pl.ds(wid * (4 * _RPW), 4 * _RPW)], prop_v)

    lane = lax.iota(jnp.int32, 16)
    for cc in range(2):
        x1 = prop_v[pl.ds(0 * _RPW + cc * 16, 16)]
        y1 = prop_v[pl.ds(1 * _RPW + cc * 16, 16)]
        x2 = prop_v[pl.ds(2 * _RPW + cc * 16, 16)]
        y2 = prop_v[pl.ds(3 * _RPW + cc * 16, 16)]
        r0 = _rint(x1 * _SCALE)
        r1 = _rint(y1 * _SCALE)
        r2 = _rint(x2 * _SCALE)
        r3 = _rint(y2 * _SCALE)
        w7 = jnp.maximum(r2 - r0 + 1.0, 1.0) / 7.0
        h7 = jnp.maximum(r3 - r1 + 1.0, 1.0) / 7.0
        xi = []
        yb = []
        for k in range(_PS):
            g = (k + 0.5) / 2.0
            xs = r0 + g * w7
            xi.append(jnp.minimum(xs.astype(jnp.int32), _W - 1))
            ys = r1 + g * h7
            yb.append(jnp.minimum(ys.astype(jnp.int32), _H - 1) * _W)
        roi_off = (lane + cc * 16) * _NR
        for ky in range(_PS):
            for kx in range(_PS):
                flat = yb[ky] + xi[kx]
                plsc.store_scatter(idx_v, [roi_off + (ky * _PS + kx)], flat)
        pad = yb[_PS - 1] + xi[_PS - 1]
        for pp in range(_PS * _PS, _NR):
            plsc.store_scatter(idx_v, [roi_off + pp], pad)

    def issue_gather(i, rows_b, sem_b):
        off = pl.multiple_of(i * _NR, 8)
        pltpu.async_copy(table_hbm.at[idx_v.at[pl.ds(off, _G1)]],
                         rows_b.at[pl.ds(0, _G1)], sem_b)
        off2 = pl.multiple_of(i * _NR + _G1, 8)
        pltpu.async_copy(table_hbm.at[idx_v.at[pl.ds(off2, _G2)]],
                         rows_b.at[pl.ds(_G1, _G2)], sem_b)

    issue_gather(0, rows0, gs0)
    issue_gather(1, rows1, gs1)

    def process(g, i, rows_b, pooled_b, gs_b, ws_b):
        pltpu.make_async_copy(table_hbm.at[pl.ds(0, _NR)], rows_b, gs_b).wait()

        @pl.when(g > 0)
        def _():
            pltpu.make_async_copy(pooled_b, out_hbm.at[:, 0], ws_b).wait()

        def pool_body(j, c2):
            py = j // _P
            px = j % _P
            r00 = py * (2 * _PS) + px * 2
            for ch in range(_CP // 16):
                s = ch * 16
                a = plsc.bitcast(rows_b[r00, pl.ds(s, 16)], jnp.bfloat16)
                b = plsc.bitcast(rows_b[r00 + 1, pl.ds(s, 16)], jnp.bfloat16)
                c = plsc.bitcast(rows_b[r00 + _PS, pl.ds(s, 16)],
                                 jnp.bfloat16)
                d = plsc.bitcast(rows_b[r00 + _PS + 1, pl.ds(s, 16)],
                                 jnp.bfloat16)
                m = jnp.maximum(jnp.maximum(a, b), jnp.maximum(c, d))
                pooled_b[j, 0, pl.ds(s, 16)] = plsc.bitcast(m, jnp.int32)
            return c2

        lax.fori_loop(0, _NB, pool_body, 0)
        pltpu.async_copy(pooled_b, out_hbm.at[:, base + i], ws_b)

        @pl.when(i + 2 < _RPW)
        def _():
            issue_gather(i + 2, rows_b, gs_b)

    def roi_pair(g, carry):
        process(g, 2 * g + 0, rows0, pooled0, gs0, ws0)
        process(g, 2 * g + 1, rows1, pooled1, gs1, ws1)
        return carry

    lax.fori_loop(0, _RPW // 2, roi_pair, 0)
    pltpu.make_async_copy(pooled0, out_hbm.at[:, 0], ws0).wait()
    pltpu.make_async_copy(pooled1, out_hbm.at[:, 0], ws1).wait()


@functools.partial(jax.jit)
def _sc_pool(table, prop_w):
    mesh = plsc.VectorSubcoreMesh(core_axis_name="c", subcore_axis_name="s")
    fn = pl.kernel(
        _sc_pool_body,
        out_type=jax.ShapeDtypeStruct((_NB, _NPAD, 1, _CP), jnp.int32),
        mesh=mesh,
        scratch_types=[
            pltpu.VMEM((4 * _RPW,), jnp.float32),
            pltpu.VMEM((_RPW * _NR,), jnp.int32),
            pltpu.VMEM((_NR, _CP), jnp.int32),
            pltpu.VMEM((_NR, _CP), jnp.int32),
            pltpu.VMEM((_NB, 1, _CP), jnp.int32),
            pltpu.VMEM((_NB, 1, _CP), jnp.int32),
            pltpu.SemaphoreType.DMA,
            pltpu.SemaphoreType.DMA,
            pltpu.SemaphoreType.DMA,
            pltpu.SemaphoreType.DMA,
        ],
        compiler_params=pltpu.CompilerParams(needs_layout_passes=False),
    )
    return fn(table, prop_w)


def _tc_body(img_ref, xs_ref, w6_ref, b6_ref, w7_ref, b7_ref, wc_ref, bc_ref,
             wr_ref, br_ref, prop_ref, cls_ref, box_ref, acc_ref):
    p = pl.program_id(0)

    @pl.when(p == 0)
    def _init():
        acc_ref[...] = jnp.zeros_like(acc_ref)

    a = xs_ref[0]
    w = w6_ref[0]
    acc_ref[...] += jnp.dot(a, w, preferred_element_type=jnp.float32)

    @pl.when(p == _NB - 1)
    def _epilogue():
        x6 = jnp.maximum(acc_ref[...] + b6_ref[...], 0.0).astype(jnp.bfloat16)
        w7 = w7_ref[...].astype(jnp.bfloat16)
        x7 = jnp.maximum(
            jnp.dot(x6, w7, preferred_element_type=jnp.float32)
            + b7_ref[...], 0.0)
        cls_ref[...] = (jnp.dot(x7, wc_ref[...],
                                preferred_element_type=jnp.float32)
                        + bc_ref[...])
        bp = (jnp.dot(x7, wr_ref[...], preferred_element_type=jnp.float32)
              + br_ref[...])
        x1 = prop_ref[:, 0:1]
        y1 = prop_ref[:, 1:2]
        x2 = prop_ref[:, 2:3]
        y2 = prop_ref[:, 3:4]
        w = x2 - x1
        h = y2 - y1
        cx = x1 + 0.5 * w
        cy = y1 + 0.5 * h
        dx = bp[:, 0 * _CPAD:1 * _CPAD]
        dy = bp[:, 1 * _CPAD:2 * _CPAD]
        dw = jnp.minimum(bp[:, 2 * _CPAD:3 * _CPAD], _LOGK)
        dh = jnp.minimum(bp[:, 3 * _CPAD:4 * _CPAD], _LOGK)
        pcx = dx * w + cx
        pcy = dy * h + cy
        pw = jnp.exp(dw) * w
        ph = jnp.exp(dh) * h
        wimg = img_ref[0, 1].astype(jnp.float32)
        himg = img_ref[0, 0].astype(jnp.float32)
        box_ref[0, :, :] = jnp.clip(pcx - 0.5 * pw, 0.0, wimg)
        box_ref[1, :, :] = jnp.clip(pcy - 0.5 * ph, 0.0, himg)
        box_ref[2, :, :] = jnp.clip(pcx + 0.5 * pw, 0.0, wimg)
        box_ref[3, :, :] = jnp.clip(pcy + 0.5 * ph, 0.0, himg)


def _tc_heads(img2, xs, w6p, b6r, w7, b7r, wc2, bc2, wr2, br2, prop):
    return pl.pallas_call(
        _tc_body,
        grid=(_NB,),
        in_specs=[
            pl.BlockSpec(memory_space=pltpu.SMEM),
            pl.BlockSpec((1, _NPAD, _C), lambda p: (p, 0, 0)),
            pl.BlockSpec((1, _C, _FC), lambda p: (p, 0, 0)),
            pl.BlockSpec((1, _FC), lambda p: (0, 0)),
            pl.BlockSpec((_FC, _FC), lambda p: (0, 0)),
            pl.BlockSpec((1, _FC), lambda p: (0, 0)),
            pl.BlockSpec((_FC, _CPAD), lambda p: (0, 0)),
            pl.BlockSpec((1, _CPAD), lambda p: (0, 0)),
            pl.BlockSpec((_FC, 4 * _CPAD), lambda p: (0, 0)),
            pl.BlockSpec((1, 4 * _CPAD), lambda p: (0, 0)),
            pl.BlockSpec((_NPAD, 4), lambda p: (0, 0)),
        ],
        out_specs=[
            pl.BlockSpec((_NPAD, _CPAD), lambda p: (0, 0)),
            pl.BlockSpec((4, _NPAD, _CPAD), lambda p: (0, 0, 0)),
        ],
        out_shape=[
            jax.ShapeDtypeStruct((_NPAD, _CPAD), jnp.float32),
            jax.ShapeDtypeStruct((4, _NPAD, _CPAD), jnp.float32),
        ],
        scratch_shapes=[pltpu.VMEM((_NPAD, _FC), jnp.float32)],
        compiler_params=pltpu.CompilerParams(
            dimension_semantics=("arbitrary",)),
    )(img2, xs, w6p, b6r, w7, b7r, wc2, bc2, wr2, br2, prop)


def kernel(feat, proposals, image_shape, W6, b6, W7, b7, Wc, bc, Wr, br):
    table = lax.bitcast_convert_type(
        _prep_table(feat.reshape(_C, _H * _W)).reshape(_H * _W, _CP, 2),
        jnp.int32)
    w6p = _prep_w6(W6.reshape(_C, _NB, 8, 128))

    prop_pad = jnp.zeros((_NPAD, 4), jnp.float32).at[:_N].set(proposals)
    prop_w = (prop_pad.reshape(_NW, _RPW, 4)
              .transpose(0, 2, 1).reshape(_NW * 4 * _RPW))

    pooled_i = _sc_pool(table, prop_w)
    xs = lax.bitcast_convert_type(pooled_i, jnp.bfloat16).reshape(
        _NB, _NPAD, _C)

    wc2 = jnp.zeros((_FC, _CPAD), jnp.float32).at[:, :_NCLS].set(Wc)
    bc2 = jnp.zeros((1, _CPAD), jnp.float32).at[0, :_NCLS].set(bc)
    wr2 = (jnp.zeros((_FC, 4, _CPAD), jnp.float32)
           .at[:, :, :_NCLS]
           .set(Wr.reshape(_FC, _NCLS, 4).transpose(0, 2, 1))
           .reshape(_FC, 4 * _CPAD))
    br2 = (jnp.zeros((1, 4, _CPAD), jnp.float32)
           .at[0, :, :_NCLS].set(br.reshape(_NCLS, 4).T)
           .reshape(1, 4 * _CPAD))
    img2 = image_shape.reshape(1, 2)

    cls_pad, box_t = _tc_heads(img2, xs,
                               w6p.reshape(_NB, _C, _FC), b6[None],
                               W7, b7[None], wc2, bc2, wr2, br2, prop_pad)
    cls_scores = cls_pad[:_N, :_NCLS]
    pred_boxes = jnp.transpose(box_t, (1, 2, 0))[:_N, :_NCLS, :]
    return cls_scores, pred_boxes


# speedup vs baseline: 1.6101x; 1.6101x over previous
import functools
import math

import jax
import jax.numpy as jnp
from jax import lax
from jax.experimental import pallas as pl
from jax.experimental.pallas import tpu as pltpu
from jax.experimental.pallas import tpu_sc as plsc

_N = 1000
_NPAD = 1024
_C = 256
_CP = _C // 2
_H = 64
_W = 64
_P = 7
_NB = _P * _P
_NCLS = 91
_CPAD = 128
_FC = 1024
_SCALE = 0.0625
_PS = 14
_NR = 200
_G1 = 96
_G2 = 104
_NW = 32
_RPW = _NPAD // _NW
_MAGIC = 8388608.0
_LOGK = math.log(1000.0 / 16)


def _rint(t):
    return (t + _MAGIC) - _MAGIC


def _table_body(feat_ref, out_ref):
    xt = jnp.transpose(feat_ref[...])
    bl = lax.bitcast_convert_type(xt[:, :_CP], jnp.int32)
    bh = lax.bitcast_convert_type(xt[:, _CP:], jnp.int32)
    half = jnp.int32(0x8000)
    word = (lax.shift_right_logical(bl + half, 16)
            | ((bh + half) & jnp.int32(-65536)))
    out_ref[...] = word


def _prep_table(feat):
    return pl.pallas_call(
        _table_body,
        out_shape=jax.ShapeDtypeStruct((_H * _W, _CP), jnp.int32),
    )(feat)


def _w6_body(w_ref, out_ref):
    out_ref[...] = w_ref[...].reshape(1, _C, 8, 128).astype(jnp.bfloat16)


def _prep_w6(w6v):
    return pl.pallas_call(
        _w6_body,
        grid=(_NB,),
        in_specs=[pl.BlockSpec((_C, 1, 8, 128), lambda p: (0, p, 0, 0))],
        out_specs=pl.BlockSpec((1, _C, 8, 128), lambda p: (p, 0, 0, 0)),
        out_shape=jax.ShapeDtypeStruct((_NB, _C, 8, 128), jnp.bfloat16),
        compiler_params=pltpu.CompilerParams(
            dimension_semantics=("arbitrary",)),
    )(w6v)


def _sc_pool_body(table_hbm, prop_hbm, out_hbm, prop_v, idx_v,
                  rows0, rows1, pooled0, pooled1, gs0, gs1, ws0, ws1):
    cid = lax.axis_index("c")
    sid = lax.axis_index("s")
    wid = sid * 2 + cid
    base = wid * _RPW

    pltpu.sync_copy(prop_hbm.at[pl.ds(wid * (4 * _RPW), 4 * _RPW)], prop_v)

    lane = lax.iota(jnp.int32, 16)
    for cc in range(2):
        x1 = prop_v[pl.ds(0 * _RPW + cc * 16, 16)]
        y1 = prop_v[pl.ds(1 * _RPW + cc * 16, 16)]
        x2 = prop_v[pl.ds(2 * _RPW + cc * 16, 16)]
        y2 = prop_v[pl.ds(3 * _RPW + cc * 16, 16)]
        r0 = _rint(x1 * _SCALE)
        r1 = _rint(y1 * _SCALE)
        r2 = _rint(x2 * _SCALE)
        r3 = _rint(y2 * _SCALE)
        w7 = jnp.maximum(r2 - r0 + 1.0, 1.0) / 7.0
        h7 = jnp.maximum(r3 - r1 + 1.0, 1.0) / 7.0
        xi = []
        yb = []
        for k in range(_PS):
            g = (k + 0.5) / 2.0
            xs = r0 + g * w7
            xi.append(jnp.minimum(xs.astype(jnp.int32), _W - 1))
            ys = r1 + g * h7
            yb.append(jnp.minimum(ys.astype(jnp.int32), _H - 1) * _W)
        roi_off = (lane + cc * 16) * _NR
        for ky in range(_PS):
            for kx in range(_PS):
                flat = yb[ky] + xi[kx]
                plsc.store_scatter(idx_v, [roi_off + (ky * _PS + kx)], flat)
        pad = yb[_PS - 1] + xi[_PS - 1]
        for pp in range(_PS * _PS, _NR):
            plsc.store_scatter(idx_v, [roi_off + pp], pad)

    def issue_gather(i, rows_b, sem_b):
        off = pl.multiple_of(i * _NR, 8)
        pltpu.async_copy(table_hbm.at[idx_v.at[pl.ds(off, _G1)]],
                         rows_b.at[pl.ds(0, _G1)], sem_b)
        off2 = pl.multiple_of(i * _NR + _G1, 8)
        pltpu.async_copy(table_hbm.at[idx_v.at[pl.ds(off2, _G2)]],
                         rows_b.at[pl.ds(_G1, _G2)], sem_b)

    issue_gather(0, rows0, gs0)
    issue_gather(1, rows1, gs1)

    def process(g, i, rows_b, pooled_b, gs_b, ws_b):
        pltpu.make_async_copy(table_hbm.at[pl.ds(0, _NR)], rows_b, gs_b).wait()

        @pl.when(g > 0)
        def _():
            pltpu.make_async_copy(pooled_b, out_hbm.at[0], ws_b).wait()

        def pool_body(j, c2):
            py = j // _P
            px = j % _P
            r00 = py * (2 * _PS) + px * 2
            for ch in range(_CP // 16):
                s = ch * 16
                a = plsc.bitcast(rows_b[r00, pl.ds(s, 16)], jnp.bfloat16)
                b = plsc.bitcast(rows_b[r00 + 1, pl.ds(s, 16)], jnp.bfloat16)
                c = plsc.bitcast(rows_b[r00 + _PS, pl.ds(s, 16)],
                                 jnp.bfloat16)
                d = plsc.bitcast(rows_b[r00 + _PS + 1, pl.ds(s, 16)],
                                 jnp.bfloat16)
                m = jnp.maximum(jnp.maximum(a, b), jnp.maximum(c, d))
                pooled_b[j, 0, pl.ds(s, 16)] = plsc.bitcast(m, jnp.int32)
            return c2

        lax.fori_loop(0, _NB, pool_body, 0)
        pltpu.async_copy(pooled_b, out_hbm.at[base + i], ws_b)

        @pl.when(i + 2 < _RPW)
        def _():
            issue_gather(i + 2, rows_b, gs_b)

    def roi_pair(g, carry):
        process(g, 2 * g + 0, rows0, pooled0, gs0, ws0)
        process(g, 2 * g + 1, rows1, pooled1, gs1, ws1)
        return carry

    lax.fori_loop(0, _RPW // 2, roi_pair, 0)
    pltpu.make_async_copy(pooled0, out_hbm.at[0], ws0).wait()
    pltpu.make_async_copy(pooled1, out_hbm.at[0], ws1).wait()


@functools.partial(jax.jit)
def _sc_pool(table, prop_w):
    mesh = plsc.VectorSubcoreMesh(core_axis_name="c", subcore_axis_name="s")
    fn = pl.kernel(
        _sc_pool_body,
        out_type=jax.ShapeDtypeStruct((_NPAD, _NB, 1, _CP), jnp.int32),
        mesh=mesh,
        scratch_types=[
            pltpu.VMEM((4 * _RPW,), jnp.float32),
            pltpu.VMEM((_RPW * _NR,), jnp.int32),
            pltpu.VMEM((_NR, _CP), jnp.int32),
            pltpu.VMEM((_NR, _CP), jnp.int32),
            pltpu.VMEM((_NB, 1, _CP), jnp.int32),
            pltpu.VMEM((_NB, 1, _CP), jnp.int32),
            pltpu.SemaphoreType.DMA,
            pltpu.SemaphoreType.DMA,
            pltpu.SemaphoreType.DMA,
            pltpu.SemaphoreType.DMA,
        ],
        compiler_params=pltpu.CompilerParams(needs_layout_passes=False),
    )
    return fn(table, prop_w)


def _tc_body(img_ref, xs_ref, w6_ref, b6_ref, w7_ref, b7_ref, wc_ref, bc_ref,
             wr_ref, br_ref, prop_ref, cls_ref, box_ref, acc_ref):
    p = pl.program_id(0)

    @pl.when(p == 0)
    def _init():
        acc_ref[...] = jnp.zeros_like(acc_ref)

    ai = xs_ref[:, 0, 0, :]
    lo = lax.bitcast_convert_type(lax.shift_left(ai, 16), jnp.float32)
    hi = lax.bitcast_convert_type(ai & jnp.int32(-65536), jnp.float32)
    a = jnp.concatenate([lo, hi], axis=1).astype(jnp.bfloat16)
    w = w6_ref[0]
    acc_ref[...] += jnp.dot(a, w, preferred_element_type=jnp.float32)

    @pl.when(p == _NB - 1)
    def _epilogue():
        x6 = jnp.maximum(acc_ref[...] + b6_ref[...], 0.0).astype(jnp.bfloat16)
        w7 = w7_ref[...].astype(jnp.bfloat16)
        x7 = jnp.maximum(
            jnp.dot(x6, w7, preferred_element_type=jnp.float32)
            + b7_ref[...], 0.0)
        cls_ref[...] = (jnp.dot(x7, wc_ref[...],
                                preferred_element_type=jnp.float32)
                        + bc_ref[...])
        bp = (jnp.dot(x7, wr_ref[...], preferred_element_type=jnp.float32)
              + br_ref[...])
        x1 = prop_ref[:, 0:1]
        y1 = prop_ref[:, 1:2]
        x2 = prop_ref[:, 2:3]
        y2 = prop_ref[:, 3:4]
        w = x2 - x1
        h = y2 - y1
        cx = x1 + 0.5 * w
        cy = y1 + 0.5 * h
        dx = bp[:, 0 * _CPAD:1 * _CPAD]
        dy = bp[:, 1 * _CPAD:2 * _CPAD]
        dw = jnp.minimum(bp[:, 2 * _CPAD:3 * _CPAD], _LOGK)
        dh = jnp.minimum(bp[:, 3 * _CPAD:4 * _CPAD], _LOGK)
        pcx = dx * w + cx
        pcy = dy * h + cy
        pw = jnp.exp(dw) * w
        ph = jnp.exp(dh) * h
        wimg = img_ref[0, 1].astype(jnp.float32)
        himg = img_ref[0, 0].astype(jnp.float32)
        box_ref[0, :, :] = jnp.clip(pcx - 0.5 * pw, 0.0, wimg)
        box_ref[1, :, :] = jnp.clip(pcy - 0.5 * ph, 0.0, himg)
        box_ref[2, :, :] = jnp.clip(pcx + 0.5 * pw, 0.0, wimg)
        box_ref[3, :, :] = jnp.clip(pcy + 0.5 * ph, 0.0, himg)


def _tc_heads(img2, xs, w6p, b6r, w7, b7r, wc2, bc2, wr2, br2, prop):
    return pl.pallas_call(
        _tc_body,
        grid=(_NB,),
        in_specs=[
            pl.BlockSpec(memory_space=pltpu.SMEM),
            pl.BlockSpec((_NPAD, 1, 1, _CP), lambda p: (0, p, 0, 0)),
            pl.BlockSpec((1, _C, _FC), lambda p: (p, 0, 0)),
            pl.BlockSpec((1, _FC), lambda p: (0, 0)),
            pl.BlockSpec((_FC, _FC), lambda p: (0, 0)),
            pl.BlockSpec((1, _FC), lambda p: (0, 0)),
            pl.BlockSpec((_FC, _CPAD), lambda p: (0, 0)),
            pl.BlockSpec((1, _CPAD), lambda p: (0, 0)),
            pl.BlockSpec((_FC, 4 * _CPAD), lambda p: (0, 0)),
            pl.BlockSpec((1, 4 * _CPAD), lambda p: (0, 0)),
            pl.BlockSpec((_NPAD, 4), lambda p: (0, 0)),
        ],
        out_specs=[
            pl.BlockSpec((_NPAD, _CPAD), lambda p: (0, 0)),
            pl.BlockSpec((4, _NPAD, _CPAD), lambda p: (0, 0, 0)),
        ],
        out_shape=[
            jax.ShapeDtypeStruct((_NPAD, _CPAD), jnp.float32),
            jax.ShapeDtypeStruct((4, _NPAD, _CPAD), jnp.float32),
        ],
        scratch_shapes=[pltpu.VMEM((_NPAD, _FC), jnp.float32)],
        compiler_params=pltpu.CompilerParams(
            dimension_semantics=("arbitrary",)),
    )(img2, xs, w6p, b6r, w7, b7r, wc2, bc2, wr2, br2, prop)


def kernel(feat, proposals, image_shape, W6, b6, W7, b7, Wc, bc, Wr, br):
    table = _prep_table(feat.reshape(_C, _H * _W))
    w6p = _prep_w6(W6.reshape(_C, _NB, 8, 128))

    prop_pad = jnp.zeros((_NPAD, 4), jnp.float32).at[:_N].set(proposals)
    prop_w = (prop_pad.reshape(_NW, _RPW, 4)
              .transpose(0, 2, 1).reshape(_NW * 4 * _RPW))

    xs = _sc_pool(table, prop_w)

    wc2 = jnp.zeros((_FC, _CPAD), jnp.float32).at[:, :_NCLS].set(Wc)
    bc2 = jnp.zeros((1, _CPAD), jnp.float32).at[0, :_NCLS].set(bc)
    wr2 = (jnp.zeros((_FC, 4, _CPAD), jnp.float32)
           .at[:, :, :_NCLS]
           .set(Wr.reshape(_FC, _NCLS, 4).transpose(0, 2, 1))
           .reshape(_FC, 4 * _CPAD))
    br2 = (jnp.zeros((1, 4, _CPAD), jnp.float32)
           .at[0, :, :_NCLS].set(br.reshape(_NCLS, 4).T)
           .reshape(1, 4 * _CPAD))
    img2 = image_shape.reshape(1, 2)

    cls_pad, box_t = _tc_heads(img2, xs,
                               w6p.reshape(_NB, _C, _FC), b6[None],
                               W7, b7[None], wc2, bc2, wr2, br2, prop_pad)
    cls_scores = cls_pad[:_N, :_NCLS]
    pred_boxes = jnp.transpose(box_t, (1, 2, 0))[:_N, :_NCLS, :]
    return cls_scores, pred_boxes
